# baseline (device time: 42337 ns/iter reference)
import jax
import jax.numpy as jnp
from jax import lax
from jax.experimental import pallas as pl
from jax.experimental.pallas import tpu as pltpu

P = 8
U = 4


def kernel(x):
    m, n = x.shape
    q = m // P
    h = q // U

    def body(x_ref, out_ref, raw_buf, send_sems, recv_sems):
        my_x = lax.axis_index("x")
        my_y = lax.axis_index("y")
        my_z = lax.axis_index("z")
        zp = my_z % 2
        me = 4 * my_x + 2 * my_y + zp

        def dev(rel):
            dx = (rel >> 2) & 1
            dy = (rel >> 1) & 1
            dz = rel & 1
            return (
                my_x + dx * (1 - 2 * my_x),
                my_y + dy * (1 - 2 * my_y),
                my_z + dz * (1 - 2 * zp),
            )

        n_x, n_y, n_z = dev(4), dev(2), dev(1)

        def rows(part, u):
            return pl.ds(part * q + u * h, h)

        barrier_sem = pltpu.get_barrier_semaphore()
        for rel in range(1, 8):
            pl.semaphore_signal(
                barrier_sem, inc=1, device_id=dev(rel),
                device_id_type=pl.DeviceIdType.MESH,
            )
        pl.semaphore_wait(barrier_sem, 7)

        sends = []

        def send(dst_dev, src, dst, recv_slot):
            rdma = pltpu.make_async_remote_copy(
                src_ref=src, dst_ref=dst,
                send_sem=send_sems.at[len(sends)],
                recv_sem=recv_sems.at[recv_slot],
                device_id=dst_dev,
                device_id_type=pl.DeviceIdType.MESH,
            )
            rdma.start()
            sends.append(rdma)

        def recv_wait(recv_slot):
            dummy = out_ref.at[pl.ds(0, h), :]
            rdma = pltpu.make_async_remote_copy(
                src_ref=dummy, dst_ref=dummy,
                send_sem=send_sems.at[0],
                recv_sem=recv_sems.at[recv_slot],
                device_id=n_x,
                device_id_type=pl.DeviceIdType.MESH,
            )
            rdma.wait_recv()


        for u in range(U):
            send(n_y, x_ref.at[rows(me ^ 2, u), :], raw_buf.at[u], u)

        for u in range(U):
            recv_wait(u)
            r = rows(me, u)
            out_ref[r, :] = x_ref[r, :] + raw_buf[u, :, :]
            o = out_ref.at[r, :]
            send(dev(4), o, o, 4 + u)
            send(dev(2), o, o, 8 + u)
            send(dev(1), o, o, 12 + u)
            send(dev(3), o, o, 16 + u)
            if u >= 1:
                send(dev(6), o, o, 20 + (u - 1))
            if u >= 2:
                send(dev(7), o, o, 28 + (u - 2))

        for u in range(U):
            recv_wait(8 + u)
            if u == 0:
                ry = out_ref.at[rows(me ^ 2, 0), :]
                send(n_x, ry, ry, 23)
            recv_wait(12 + u)
            rz = out_ref.at[rows(me ^ 1, u), :]
            send(n_x, rz, rz, 24 + u)
            recv_wait(16 + u)
            if u <= 1:
                rd = out_ref.at[rows(me ^ 3, u), :]
                send(n_x, rd, rd, 30 + u)

        for s in list(range(4, 8)) + list(range(20, 32)):
            recv_wait(s)
        for s in sends:
            s.wait_send()

    return pl.pallas_call(
        body,
        out_shape=jax.ShapeDtypeStruct((m, n), x.dtype),
        in_specs=[pl.BlockSpec(memory_space=pltpu.VMEM)],
        out_specs=pl.BlockSpec(memory_space=pltpu.VMEM),
        scratch_shapes=[
            pltpu.VMEM((U, h, n), x.dtype),
            pltpu.SemaphoreType.DMA((32,)),
            pltpu.SemaphoreType.DMA((32,)),
        ],
        compiler_params=pltpu.CompilerParams(collective_id=0),
    )(x)


# device time: 29867 ns/iter; 1.4175x vs baseline; 1.4175x over previous
import jax
import jax.numpy as jnp
from jax import lax
from jax.experimental import pallas as pl
from jax.experimental.pallas import tpu as pltpu

P = 8
U = 4


def kernel(x):
    m, n = x.shape
    q = m // P
    h = q // U

    def body(x_ref, out_ref, raw_buf, send_sems, recv_sems):
        my_x = lax.axis_index("x")
        my_y = lax.axis_index("y")
        my_z = lax.axis_index("z")
        zp = my_z % 2
        n_x = (1 - my_x, my_y, my_z)
        n_y = (my_x, 1 - my_y, my_z)
        n_z = (my_x, my_y, my_z + 1 - 2 * zp)
        me = 4 * my_x + 2 * my_y + zp

        def rows(part, u):
            return pl.ds(part * q + u * h, h)

        barrier_sem = pltpu.get_barrier_semaphore()
        for nbr in (n_x, n_y, n_z):
            pl.semaphore_signal(
                barrier_sem, inc=1, device_id=nbr,
                device_id_type=pl.DeviceIdType.MESH,
            )
        pl.semaphore_wait(barrier_sem, 3)

        sends = []

        def send(dst_dev, src, dst, recv_slot):
            rdma = pltpu.make_async_remote_copy(
                src_ref=src, dst_ref=dst,
                send_sem=send_sems.at[len(sends)],
                recv_sem=recv_sems.at[recv_slot],
                device_id=dst_dev,
                device_id_type=pl.DeviceIdType.MESH,
            )
            rdma.start()
            sends.append(rdma)

        def recv_wait(dst, recv_slot):
            rdma = pltpu.make_async_remote_copy(
                src_ref=dst, dst_ref=dst,
                send_sem=send_sems.at[0],
                recv_sem=recv_sems.at[recv_slot],
                device_id=n_x,
                device_id_type=pl.DeviceIdType.MESH,
            )
            rdma.wait_recv()

        for u in range(U):
            send(n_y, x_ref.at[rows(me ^ 2, u), :], raw_buf.at[u], u)

        for u in range(U):
            recv_wait(raw_buf.at[u], u)
            r = rows(me, u)
            out_ref[r, :] = x_ref[r, :] + raw_buf[u, :, :]
            send(n_x, out_ref.at[r, :], out_ref.at[r, :], 4 + u)
            send(n_y, out_ref.at[r, :], out_ref.at[r, :], 8 + u)
            send(n_z, out_ref.at[r, :], out_ref.at[r, :], 12 + u)

        for u in range(U):
            rx = rows(me ^ 4, u)
            recv_wait(out_ref.at[rx, :], 4 + u)
            if u <= 2:
                send(n_y, out_ref.at[rx, :], out_ref.at[rx, :], 24 + u)
            ry = rows(me ^ 2, u)
            recv_wait(out_ref.at[ry, :], 8 + u)
            send(n_z, out_ref.at[ry, :], out_ref.at[ry, :], 20 + u)
            if u == 3:
                send(n_x, out_ref.at[ry, :], out_ref.at[ry, :], 27)
            rz = rows(me ^ 1, u)
            recv_wait(out_ref.at[rz, :], 12 + u)
            send(n_x, out_ref.at[rz, :], out_ref.at[rz, :], 16 + u)

        for u in range(U):
            recv_wait(out_ref.at[rows(me ^ 5, u), :], 16 + u)
            rb = rows(me ^ 3, u)
            recv_wait(out_ref.at[rb, :], 20 + u)
            if u <= 1:
                send(n_x, out_ref.at[rb, :], out_ref.at[rb, :], 28 + u)
            rc = rows(me ^ 6, u)
            if u <= 2:
                recv_wait(out_ref.at[rc, :], 24 + u)
                if u == 2:
                    send(n_z, out_ref.at[rc, :], out_ref.at[rc, :], 30)
            else:
                recv_wait(out_ref.at[rc, :], 27)
                send(n_z, out_ref.at[rc, :], out_ref.at[rc, :], 31)

        for u in range(U):
            recv_wait(out_ref.at[rows(me ^ 7, u), :], 28 + u)

        for s in sends:
            s.wait_send()

    return pl.pallas_call(
        body,
        out_shape=jax.ShapeDtypeStruct((m, n), x.dtype),
        in_specs=[pl.BlockSpec(memory_space=pltpu.VMEM)],
        out_specs=pl.BlockSpec(memory_space=pltpu.VMEM),
        scratch_shapes=[
            pltpu.VMEM((U, h, n), x.dtype),
            pltpu.SemaphoreType.DMA((32,)),
            pltpu.SemaphoreType.DMA((32,)),
        ],
        compiler_params=pltpu.CompilerParams(collective_id=0),
    )(x)
